# baseline (speedup 1.0000x reference)
import functools

import jax
import jax.numpy as jnp
from jax import lax
from jax.experimental import pallas as pl
from jax.experimental.pallas import tpu as pltpu
from jax.experimental.pallas import tpu_sc as plsc

N = 10000
E = 320000
D = 128
DH = 64
NC = 2
NS = 16
BATCH = 128
NB_S = 160
EP = NS * NB_S * BATCH
NP = 10240
RPT = NP // NS
R_BUF = 4
G_AHEAD = 2


def _sc_agg_body(with_count, *refs):
    n_in = 5 if with_count else 4
    if with_count:
        x2_hbm, src_hbm, dst_hbm, acc_out, cnt_out = refs[:5]
    else:
        x2_hbm, src_hbm, dst_hbm, acc_out = refs[:4]
    rest = list(refs[n_in:])
    src_v, dst_v = rest[0], rest[1]
    bufs = rest[2:2 + R_BUF]
    k = 2 + R_BUF
    if with_count:
        ones_v = rest[k]
        k += 1
    acc_sh = rest[k]
    k += 1
    if with_count:
        cnt_sh = rest[k]
        k += 1
    sem_g = rest[k:k + R_BUF]
    sem_s = rest[k + R_BUF:k + 2 * R_BUF]

    c = lax.axis_index("c")
    s = lax.axis_index("s")
    row0 = s * RPT

    z_v = bufs[0]

    @pl.loop(0, BATCH)
    def _(i):
        @pl.loop(0, DH, step=16)
        def _(kk):
            z_v[i, pl.ds(kk, 16)] = jnp.zeros((16,), jnp.float32)

    @pl.loop(0, RPT, step=BATCH)
    def _(r):
        pltpu.sync_copy(z_v, acc_sh.at[pl.ds(row0 + r, BATCH)])

    if with_count:
        @pl.loop(0, BATCH)
        def _(i):
            ones_v[i, :] = jnp.zeros((16,), jnp.float32)

        @pl.loop(0, RPT, step=BATCH)
        def _(r):
            pltpu.sync_copy(ones_v, cnt_sh.at[pl.ds(row0 + r, BATCH)])

        @pl.loop(0, BATCH)
        def _(i):
            ones_v[i, :] = jnp.ones((16,), jnp.float32)

    pltpu.sync_copy(src_hbm.at[pl.ds(s * NB_S, NB_S)], src_v)
    pltpu.sync_copy(dst_hbm.at[pl.ds(s * NB_S, NB_S)], dst_v)

    plsc.subcore_barrier()

    def gat(i, b):
        pltpu.async_copy(x2_hbm.at[c].at[src_v.at[i]], bufs[b], sem_g[b])

    def gat_wait(b):
        pltpu.make_async_copy(
            x2_hbm.at[c].at[src_v.at[0]], bufs[b], sem_g[b]).wait()

    def scat(i, b, parity):
        pltpu.async_copy(bufs[b], acc_sh.at[dst_v.at[i]], sem_s[b], add=True)
        if with_count:
            @pl.when(c == parity)
            def _():
                pltpu.sync_copy(ones_v, cnt_sh.at[dst_v.at[i]], add=True)

    def scat_wait(b):
        pltpu.make_async_copy(
            bufs[b], acc_sh.at[dst_v.at[0]], sem_s[b]).wait()

    for b in range(G_AHEAD):
        gat(b, b)

    for b in range(R_BUF):
        gat_wait(b)
        scat(b, b, b % 2)
        t = (b + G_AHEAD) % R_BUF
        if b >= G_AHEAD:
            scat_wait(t)
        gat(b + G_AHEAD, t)

    @pl.loop(1, NB_S // R_BUF - 1)
    def _(kk):
        base = kk * R_BUF
        for b in range(R_BUF):
            i = base + b
            gat_wait(b)
            scat(i, b, b % 2)
            t = (b + G_AHEAD) % R_BUF
            scat_wait(t)
            gat(i + G_AHEAD, t)

    base = NB_S - R_BUF
    for b in range(G_AHEAD):
        i = base + b
        gat_wait(b)
        scat(i, b, b % 2)
        t = b + G_AHEAD
        scat_wait(t)
        gat(i + G_AHEAD, t)
    for b in range(G_AHEAD, R_BUF):
        gat_wait(b)
        scat(base + b, b, b % 2)
    for b in range(R_BUF):
        scat_wait(b)

    plsc.subcore_barrier()

    pltpu.sync_copy(acc_sh.at[pl.ds(row0, RPT)], acc_out.at[c, pl.ds(row0, RPT)])
    if with_count:
        pltpu.sync_copy(cnt_sh.at[pl.ds(row0, RPT)],
                        cnt_out.at[c, pl.ds(row0, RPT)])


def _make_sc_agg(with_count):
    mesh = plsc.VectorSubcoreMesh(core_axis_name="c", subcore_axis_name="s")
    out_type = [jax.ShapeDtypeStruct((NC, NP, DH), jnp.float32)]
    scratch = [
        pltpu.VMEM((NB_S, BATCH), jnp.int32),
        pltpu.VMEM((NB_S, BATCH), jnp.int32),
    ]
    scratch += [pltpu.VMEM((BATCH, DH), jnp.float32) for _ in range(R_BUF)]
    if with_count:
        out_type.append(jax.ShapeDtypeStruct((NC, NP, 16), jnp.float32))
        scratch.append(pltpu.VMEM((BATCH, 16), jnp.float32))
    scratch.append(pltpu.VMEM_SHARED((NP, DH), jnp.float32))
    if with_count:
        scratch.append(pltpu.VMEM_SHARED((NP, 16), jnp.float32))
    scratch += [pltpu.SemaphoreType.DMA for _ in range(2 * R_BUF)]
    return pl.kernel(
        functools.partial(_sc_agg_body, with_count),
        out_type=out_type,
        mesh=mesh,
        scratch_types=scratch,
        compiler_params=pltpu.CompilerParams(use_tc_tiling_on_sc=False),
    )


def _tc_layer_body(relu, acc_ref, cnt_ref, x_ref, wl_ref, wr_ref, b_ref, o_ref):
    agg = jnp.concatenate([acc_ref[0, :N, :], acc_ref[1, :N, :]], axis=1)
    if x_ref.shape[0] == NC:
        xin = jnp.concatenate([x_ref[0], x_ref[1]], axis=1)
    else:
        xin = x_ref[...]
    cnt = cnt_ref[0, :N, 0:1] + cnt_ref[1, :N, 0:1]
    mean = agg / jnp.maximum(cnt, 1.0)
    r = (jnp.dot(mean, wl_ref[...], preferred_element_type=jnp.float32)
         + jnp.dot(xin, wr_ref[...], preferred_element_type=jnp.float32)
         + b_ref[...])
    r = jnp.maximum(r, 0.0) if relu else r
    if o_ref.shape[0] == NC:
        o_ref[0] = r[:, :DH]
        o_ref[1] = r[:, DH:]
    else:
        o_ref[...] = r


def _make_tc_layer(relu, split_out):
    out_shape = (jax.ShapeDtypeStruct((NC, N, DH), jnp.float32) if split_out
                 else jax.ShapeDtypeStruct((N, D), jnp.float32))
    return pl.pallas_call(
        functools.partial(_tc_layer_body, relu),
        out_shape=out_shape,
    )


_sc_agg_cnt = _make_sc_agg(True)
_sc_agg = _make_sc_agg(False)
_tc_layer1 = _make_tc_layer(True, True)
_tc_layer2 = _make_tc_layer(False, False)


def kernel(x, edge_index, W1_l, W1_r, b1, W2_l, W2_r, b2):
    src = edge_index[0].astype(jnp.int32)
    dst = edge_index[1].astype(jnp.int32)
    pad = EP - E
    src_p = jnp.concatenate([src, jnp.zeros((pad,), jnp.int32)])
    dst_p = jnp.concatenate(
        [dst, N + (jnp.arange(pad, dtype=jnp.int32) % (NP - N))])
    src_p = src_p.reshape(EP // BATCH, BATCH)
    dst_p = dst_p.reshape(EP // BATCH, BATCH)
    x2 = x.reshape(N, NC, DH).transpose(1, 0, 2)
    src_p, dst_p, x2 = lax.optimization_barrier((src_p, dst_p, x2))

    acc1, cnt1 = _sc_agg_cnt(x2, src_p, dst_p)
    h2 = _tc_layer1(acc1, cnt1, x, W1_l, W1_r, b1.reshape(1, D))
    (acc2,) = _sc_agg(h2, src_p, dst_p)
    out = _tc_layer2(acc2, cnt1, h2, W2_l, W2_r, b2.reshape(1, D))
    return out


# speedup vs baseline: 1.7949x; 1.7949x over previous
import functools

import jax
import jax.numpy as jnp
from jax import lax
from jax.experimental import pallas as pl
from jax.experimental.pallas import tpu as pltpu
from jax.experimental.pallas import tpu_sc as plsc

N = 10000
E = 320000
D = 128
DH = 64
NC = 2
NS = 16
BATCH = 64
NB_S = 320
EP = NS * NB_S * BATCH
NP = 10240
RPT = NP // NS
XRT = 624
CW = 8


def _sc_agg_body(x2_hbm, src_hbm, dst_hbm, acc_out,
                 src_v, dst_v, buf0, buf1, acc_sh, x_sp, sem0, sem1):
    c = lax.axis_index("c")
    s = lax.axis_index("s")
    row0 = s * RPT

    @pl.loop(0, BATCH)
    def _(i):
        @pl.loop(0, DH, step=16)
        def _(kk):
            buf0[i, pl.ds(kk, 16)] = jnp.zeros((16,), jnp.float32)

    @pl.loop(0, RPT, step=BATCH)
    def _(r):
        pltpu.sync_copy(buf0, acc_sh.at[pl.ds(row0 + r, BATCH)])

    pltpu.sync_copy(x2_hbm.at[c].at[pl.ds(s * XRT, XRT)],
                    x_sp.at[pl.ds(s * XRT, XRT)])

    @pl.when(s == NS - 1)
    def _():
        pltpu.sync_copy(x2_hbm.at[c].at[pl.ds(XRT * NS, N - XRT * NS)],
                        x_sp.at[pl.ds(XRT * NS, N - XRT * NS)])

    pltpu.sync_copy(src_hbm.at[pl.ds(s * NB_S, NB_S)], src_v)
    pltpu.sync_copy(dst_hbm.at[pl.ds(s * NB_S, NB_S)], dst_v)

    plsc.subcore_barrier()

    bufs = (buf0, buf1)
    sems = (sem0, sem1)

    def gat(i, b):
        pltpu.async_copy(x_sp.at[src_v.at[i]], bufs[b], sems[b])

    def gat_wait(b):
        pltpu.make_async_copy(x_sp.at[src_v.at[0]], bufs[b], sems[b]).wait()

    def scat(i, b):
        pltpu.sync_copy(bufs[b], acc_sh.at[dst_v.at[i]], add=True)

    gat(0, 0)

    @pl.loop(0, NB_S - 2, step=2)
    def _(j):
        gat(j + 1, 1)
        gat_wait(0)
        scat(j, 0)
        gat(j + 2, 0)
        gat_wait(1)
        scat(j + 1, 1)

    gat(NB_S - 1, 1)
    gat_wait(0)
    scat(NB_S - 2, 0)
    gat_wait(1)
    scat(NB_S - 1, 1)

    plsc.subcore_barrier()

    pltpu.sync_copy(acc_sh.at[pl.ds(row0, RPT)], acc_out.at[c, pl.ds(row0, RPT)])


def _make_sc_agg():
    mesh = plsc.VectorSubcoreMesh(core_axis_name="c", subcore_axis_name="s")
    scratch = [
        pltpu.VMEM((NB_S, BATCH), jnp.int32),
        pltpu.VMEM((NB_S, BATCH), jnp.int32),
        pltpu.VMEM((BATCH, DH), jnp.float32),
        pltpu.VMEM((BATCH, DH), jnp.float32),
        pltpu.VMEM_SHARED((NP, DH), jnp.float32),
        pltpu.VMEM_SHARED((N, DH), jnp.float32),
        pltpu.SemaphoreType.DMA,
        pltpu.SemaphoreType.DMA,
    ]
    return pl.kernel(
        _sc_agg_body,
        out_type=[jax.ShapeDtypeStruct((NC, NP, DH), jnp.float32)],
        mesh=mesh,
        scratch_types=scratch,
        compiler_params=pltpu.CompilerParams(use_tc_tiling_on_sc=False),
    )


def _sc_cnt_body(dst_hbm, cnt_out, dst_v, ones_v, cnt_sh, sem):
    c = lax.axis_index("c")
    s = lax.axis_index("s")
    row0 = s * RPT

    @pl.loop(0, BATCH)
    def _(i):
        ones_v[i, :] = jnp.zeros((16,), jnp.float32)

    @pl.loop(0, RPT, step=BATCH)
    def _(r):
        pltpu.sync_copy(ones_v, cnt_sh.at[pl.ds(row0 + r, BATCH)])

    @pl.loop(0, BATCH)
    def _(i):
        ones_v[i, :] = jnp.ones((16,), jnp.float32)

    pltpu.sync_copy(dst_hbm.at[pl.ds(s * NB_S, NB_S)], dst_v)

    plsc.subcore_barrier()

    def cnt_issue(j):
        pltpu.async_copy(ones_v, cnt_sh.at[dst_v.at[j]], sem, add=True)

    def cnt_wait():
        pltpu.make_async_copy(ones_v, cnt_sh.at[dst_v.at[0]], sem).wait()

    @pl.loop(0, NB_S, step=2)
    def _(j):
        cnt_issue(j + c)

        @pl.when(j >= 2 * CW)
        def _():
            cnt_wait()

    @pl.loop(0, CW)
    def _(j):
        cnt_wait()

    plsc.subcore_barrier()

    pltpu.sync_copy(cnt_sh.at[pl.ds(row0, RPT)], cnt_out.at[c, pl.ds(row0, RPT)])


def _make_sc_cnt():
    mesh = plsc.VectorSubcoreMesh(core_axis_name="c", subcore_axis_name="s")
    scratch = [
        pltpu.VMEM((NB_S, BATCH), jnp.int32),
        pltpu.VMEM((BATCH, 16), jnp.float32),
        pltpu.VMEM_SHARED((NP, 16), jnp.float32),
        pltpu.SemaphoreType.DMA,
    ]
    return pl.kernel(
        _sc_cnt_body,
        out_type=[jax.ShapeDtypeStruct((NC, NP, 16), jnp.float32)],
        mesh=mesh,
        scratch_types=scratch,
        compiler_params=pltpu.CompilerParams(use_tc_tiling_on_sc=False),
    )


def _tc_layer_body(relu, acc_ref, cnt_ref, x_ref, wl_ref, wr_ref, b_ref, o_ref):
    agg = jnp.concatenate([acc_ref[0, :N, :], acc_ref[1, :N, :]], axis=1)
    if x_ref.shape[0] == NC:
        xin = jnp.concatenate([x_ref[0], x_ref[1]], axis=1)
    else:
        xin = x_ref[...]
    cnt = cnt_ref[0, :N, 0:1] + cnt_ref[1, :N, 0:1]
    mean = agg / jnp.maximum(cnt, 1.0)
    r = (jnp.dot(mean, wl_ref[...], preferred_element_type=jnp.float32)
         + jnp.dot(xin, wr_ref[...], preferred_element_type=jnp.float32)
         + b_ref[...])
    r = jnp.maximum(r, 0.0) if relu else r
    if o_ref.shape[0] == NC:
        o_ref[0] = r[:, :DH]
        o_ref[1] = r[:, DH:]
    else:
        o_ref[...] = r


def _make_tc_layer(relu, split_out):
    out_shape = (jax.ShapeDtypeStruct((NC, N, DH), jnp.float32) if split_out
                 else jax.ShapeDtypeStruct((N, D), jnp.float32))
    return pl.pallas_call(
        functools.partial(_tc_layer_body, relu),
        out_shape=out_shape,
    )


_sc_agg = _make_sc_agg()
_sc_cnt = _make_sc_cnt()
_tc_layer1 = _make_tc_layer(True, True)
_tc_layer2 = _make_tc_layer(False, False)


def kernel(x, edge_index, W1_l, W1_r, b1, W2_l, W2_r, b2):
    src = edge_index[0].astype(jnp.int32)
    dst = edge_index[1].astype(jnp.int32)
    pad = EP - E
    src_p = jnp.concatenate([src, jnp.zeros((pad,), jnp.int32)])
    dst_p = jnp.concatenate(
        [dst, N + (jnp.arange(pad, dtype=jnp.int32) % (NP - N))])
    src_p = src_p.reshape(EP // BATCH, BATCH)
    dst_p = dst_p.reshape(EP // BATCH, BATCH)
    x2 = x.reshape(N, NC, DH).transpose(1, 0, 2)

    (cnt1,) = _sc_cnt(dst_p)
    (acc1,) = _sc_agg(x2, src_p, dst_p)
    h2 = _tc_layer1(acc1, cnt1, x, W1_l, W1_r, b1.reshape(1, D))
    (acc2,) = _sc_agg(h2, src_p, dst_p)
    out = _tc_layer2(acc2, cnt1, h2, W2_l, W2_r, b2.reshape(1, D))
    return out
